# baseline (speedup 1.0000x reference)
import jax
import jax.numpy as jnp
from jax import lax
from jax.experimental import pallas as pl
from jax.experimental.pallas import tpu as pltpu
from jax.experimental.pallas import tpu_sc as plsc

NC = 2
NS = 16
NW = NC * NS
L = 16

BATCH = 16384
ACT = 64
UNIT = 128
BPW = BATCH // NW
HALF = BPW // 2
HGROUPS = HALF // L


def _body(units_hbm, offs_hbm, table_hbm, out_hbm, units_v, offs_v, rows_v,
          buf_v, sem0, sem1):
    wid = lax.axis_index("s") * NC + lax.axis_index("c")
    base = wid * BPW

    pltpu.sync_copy(units_hbm.at[pl.ds(base, BPW)], units_v)
    pltpu.sync_copy(offs_hbm.at[pl.ds(base, BPW)], offs_v)

    sems = (sem0, sem1)
    for h in range(2):
        pltpu.async_copy(
            table_hbm.at[units_v.at[pl.ds(h * HALF, HALF)]],
            rows_v.at[h], sems[h],
        )

    lanes = lax.iota(jnp.int32, L)

    for h in range(2):
        pltpu.make_async_copy(
            table_hbm.at[units_v.at[pl.ds(h * HALF, HALF)]],
            rows_v.at[h], sems[h],
        ).wait()

        def group_body(g, carry):
            rid = g * L + lanes
            sl = pl.ds(h * HALF + g * L, L)
            off = offs_v[sl]

            m = [jnp.full((L,), -jnp.inf, jnp.float32) for _ in range(4)]
            for j in range(ACT):
                v = plsc.load_gather(rows_v.at[h], [rid, off + j])
                m[j % 4] = jnp.maximum(m[j % 4], v)
            mx = jnp.maximum(jnp.maximum(m[0], m[1]),
                             jnp.maximum(m[2], m[3]))

            s = [jnp.zeros((L,), jnp.float32) for _ in range(4)]
            for j in range(ACT):
                v = plsc.load_gather(rows_v.at[h], [rid, off + j])
                e = jnp.exp(v - mx)
                buf_v[j, sl] = e
                s[j % 4] = s[j % 4] + e
            r = 1.0 / ((s[0] + s[1]) + (s[2] + s[3]))

            for j in range(ACT):
                buf_v[j, sl] = buf_v[j, sl] * r
            return carry

        lax.fori_loop(0, HGROUPS, group_body, 0)

    pltpu.sync_copy(buf_v, out_hbm.at[:, pl.ds(base, BPW)])


def kernel(state, logits):
    state = state.astype(jnp.int32)
    units = lax.shift_right_logical(state, 1)
    offs = lax.shift_left(jnp.bitwise_and(state, 1), 6)
    n2 = logits.shape[0] // 2
    table = (
        logits.T.reshape(ACT, n2, 2).transpose(1, 2, 0).reshape(n2, 2 * ACT)
    )

    mesh = plsc.VectorSubcoreMesh(core_axis_name="c", subcore_axis_name="s")
    k = pl.kernel(
        _body,
        mesh=mesh,
        out_type=jax.ShapeDtypeStruct((ACT, BATCH), jnp.float32),
        scratch_types=[
            pltpu.VMEM((BPW,), jnp.int32),
            pltpu.VMEM((BPW,), jnp.int32),
            pltpu.VMEM((2, HALF, UNIT), jnp.float32),
            pltpu.VMEM((ACT, BPW), jnp.float32),
            pltpu.SemaphoreType.DMA,
            pltpu.SemaphoreType.DMA,
        ],
        compiler_params=pltpu.CompilerParams(needs_layout_passes=False),
    )
    return k(units, offs, table).T


# speedup vs baseline: 1.2274x; 1.2274x over previous
import jax
import jax.numpy as jnp
from jax import lax
from jax.experimental import pallas as pl
from jax.experimental.pallas import tpu as pltpu
from jax.experimental.pallas import tpu_sc as plsc

NC = 2
NS = 16
NW = NC * NS
L = 16

BATCH = 16384
ACT = 64
UNIT = 128
BPW = BATCH // NW
HALF = BPW // 2
HGROUPS = HALF // L


def _body(units_hbm, offs_hbm, table_hbm, out_hbm, units_v, offs_v, rows_v,
          buf_v, sem0, sem1):
    wid = lax.axis_index("s") * NC + lax.axis_index("c")
    base = wid * BPW

    pltpu.sync_copy(units_hbm.at[pl.ds(base, BPW)], units_v)
    pltpu.sync_copy(offs_hbm.at[pl.ds(base, BPW)], offs_v)

    sems = (sem0, sem1)
    for h in range(2):
        pltpu.async_copy(
            table_hbm.at[units_v.at[pl.ds(h * HALF, HALF)]],
            rows_v.at[h], sems[h],
        )

    lanes = lax.iota(jnp.int32, L)

    for h in range(2):
        pltpu.make_async_copy(
            table_hbm.at[units_v.at[pl.ds(h * HALF, HALF)]],
            rows_v.at[h], sems[h],
        ).wait()

        def group_body(g, carry):
            rid = g * L + lanes
            sl = pl.ds(h * HALF + g * L, L)
            off = offs_v[sl]

            m = [jnp.full((L,), -jnp.inf, jnp.float32) for _ in range(4)]
            for j in range(ACT):
                v = plsc.load_gather(rows_v.at[h], [rid, off + j])
                m[j % 4] = jnp.maximum(m[j % 4], v)
            mx = jnp.maximum(jnp.maximum(m[0], m[1]),
                             jnp.maximum(m[2], m[3]))

            s = [jnp.zeros((L,), jnp.float32) for _ in range(4)]
            for j in range(ACT):
                v = plsc.load_gather(rows_v.at[h], [rid, off + j])
                e = jnp.exp(v - mx)
                buf_v[j, sl] = e
                s[j % 4] = s[j % 4] + e
            r = 1.0 / ((s[0] + s[1]) + (s[2] + s[3]))

            for j in range(ACT):
                buf_v[j, sl] = buf_v[j, sl] * r
            return carry

        lax.fori_loop(0, HGROUPS, group_body, 0)

    pltpu.sync_copy(buf_v, out_hbm.at[:, pl.ds(base, BPW)])


def kernel(state, logits):
    state = state.astype(jnp.int32)
    units = lax.shift_right_logical(state, 1)
    offs = lax.shift_left(jnp.bitwise_and(state, 1), 6)
    n2 = logits.shape[0] // 2
    table = logits.reshape(n2, 2 * ACT)

    mesh = plsc.VectorSubcoreMesh(core_axis_name="c", subcore_axis_name="s")
    k = pl.kernel(
        _body,
        mesh=mesh,
        out_type=jax.ShapeDtypeStruct((ACT, BATCH), jnp.float32),
        scratch_types=[
            pltpu.VMEM((BPW,), jnp.int32),
            pltpu.VMEM((BPW,), jnp.int32),
            pltpu.VMEM((2, HALF, UNIT), jnp.float32),
            pltpu.VMEM((ACT, BPW), jnp.float32),
            pltpu.SemaphoreType.DMA,
            pltpu.SemaphoreType.DMA,
        ],
        compiler_params=pltpu.CompilerParams(needs_layout_passes=False),
    )
    return k(units, offs, table).T
